# Initial kernel scaffold; baseline (speedup 1.0000x reference)
import jax
import jax.numpy as jnp
from jax.experimental import pallas as pl


def kernel(image_embedding, image_pe, params, original_shape):
    raise NotImplementedError("write your pallas kernel here")


# speedup vs baseline: 1.4283x; 1.4283x over previous
import math
import functools

import numpy as np
import jax
import jax.numpy as jnp
from jax import lax
from jax.experimental import pallas as pl

_B = 2
_H = 224
_W = 224
_H0 = 64
_ROWS_PER_TILE = 8
_TILE_TOKENS = _ROWS_PER_TILE * _W
_TILES_PER_BATCH = _H // _ROWS_PER_TILE
_NUM_TILES = _B * _TILES_PER_BATCH
_K = int(_B * _H * _W * 0.2)
_FINE_TILES = -(-_K // _TILE_TOKENS)
_NUM_FREQ = 10
_MAX_FREQ = 10.0
_EPS = 1e-5
_FREQS = np.exp2(np.linspace(0.0, _MAX_FREQ, _NUM_FREQ).astype(np.float32)).astype(np.float32)


def _tile_kernel(emb_ref, pe_ref, m_ref,
                 w0a_ref, b0a_ref, w0b_ref, b0b_ref, wh_ref, bh_ref,
                 w1a_ref, b1a_ref, w1b_ref, b1b_ref,
                 w3a_ref, b3a_ref, w3b_ref, b3b_ref,
                 coarse_ref, fine_ref):
    i = pl.program_id(0)
    r0 = lax.rem(i, _TILES_PER_BATCH) * _ROWS_PER_TILE

    m_full = m_ref[...]
    m_rows = m_ref[pl.ds(r0, _ROWS_PER_TILE), :]

    def interp(img):
        t = lax.dot_general(m_rows, img, (((1,), (1,)), ((), ())),
                            preferred_element_type=jnp.float32)
        t = lax.dot_general(t, m_full, (((2,), (1,)), ((), ())),
                            preferred_element_type=jnp.float32)
        t = jnp.transpose(t, (0, 2, 1))
        return t.reshape(_TILE_TOKENS, img.shape[0])

    emb_f = interp(emb_ref[0])
    pe_f = interp(pe_ref[0])

    row_i = lax.broadcasted_iota(jnp.int32, (_ROWS_PER_TILE, _W), 0).astype(jnp.float32)
    col_i = lax.broadcasted_iota(jnp.int32, (_ROWS_PER_TILE, _W), 1).astype(jnp.float32)
    gy = -1.0 + (r0.astype(jnp.float32) + row_i) * (2.0 / (_H - 1))
    gx = -1.0 + col_i * (2.0 / (_W - 1))
    si = jnp.concatenate(
        [(2.0 * math.pi * float(f)) * gy[..., None] for f in _FREQS]
        + [(2.0 * math.pi * float(f)) * gx[..., None] for f in _FREQS], axis=-1)
    enc = jnp.concatenate([jnp.sin(si), jnp.sin(si + math.pi / 2.0),
                           gy[..., None], gx[..., None]], axis=-1)
    coords = enc.reshape(_TILE_TOKENS, 2 * _NUM_FREQ * 2 + 2)

    feat = jnp.concatenate([emb_f, pe_f, coords], axis=1)

    def ldot(x, w_ref, b_ref):
        return lax.dot_general(x, w_ref[...], (((1,), (0,)), ((), ())),
                               preferred_element_type=jnp.float32) + b_ref[...]

    h = jax.nn.relu(ldot(feat, w0a_ref, b0a_ref))
    h = jax.nn.relu(ldot(h, w0b_ref, b0b_ref))
    s = ldot(h, wh_ref, bh_ref)

    s0 = s[:, 0].reshape(1, _ROWS_PER_TILE, _W)
    coarse_ref[...] = s0

    @pl.when(i < _FINE_TILES)
    def _fine():
        fine_in = jnp.concatenate([feat, s[:, 1:]], axis=1)
        y = jax.nn.relu(ldot(fine_in, w1a_ref, b1a_ref))
        y = jax.nn.relu(ldot(y, w1b_ref, b1b_ref))
        z = jax.nn.relu(ldot(y, w3a_ref, b3a_ref))
        z = ldot(z, w3b_ref, b3b_ref)
        li = lax.broadcasted_iota(jnp.int32, (_ROWS_PER_TILE, _W), 0) * _W \
            + lax.broadcasted_iota(jnp.int32, (_ROWS_PER_TILE, _W), 1)
        tid = i * _TILE_TOKENS + li
        z2 = z.reshape(_ROWS_PER_TILE, _W)
        fine_ref[...] = jnp.where(tid < _K, z2, s0[0])[None]

    @pl.when(i >= _FINE_TILES)
    def _copy():
        fine_ref[...] = s0


def _fold(lin, bn):
    scale = bn['g'] / jnp.sqrt(bn['v'] + _EPS)
    w = lin['W'] * scale[None, :]
    b = (lin['b'] - bn['m']) * scale + bn['be']
    return w.astype(jnp.float32), b.astype(jnp.float32).reshape(1, -1)


@jax.jit
def _run(image_embedding, image_pe, params):
    p = params
    w0a, b0a = _fold(p['l0a'], p['bn0a'])
    w0b, b0b = _fold(p['l0b'], p['bn0b'])
    wh = p['head']['W'].astype(jnp.float32)
    bh = p['head']['b'].astype(jnp.float32).reshape(1, -1)
    w1a, b1a = _fold(p['l1a'], p['bn1a'])
    w1b, b1b = _fold(p['l1b'], p['bn1b'])
    w3a, b3a = _fold(p['l3a'], p['bn3a'])
    w3b = p['l3b']['W'].astype(jnp.float32)
    b3b = p['l3b']['b'].astype(jnp.float32).reshape(1, -1)

    m = jax.image.resize(jnp.eye(_H0, dtype=jnp.float32), (_H, _H0),
                         method='bilinear')

    def whole(a):
        return pl.BlockSpec(a.shape, lambda i: (0,) * a.ndim)

    emb = image_embedding.astype(jnp.float32)
    pe = image_pe.astype(jnp.float32)

    grid = (_NUM_TILES,)
    in_specs = [
        pl.BlockSpec((1,) + emb.shape[1:], lambda i: (i // _TILES_PER_BATCH, 0, 0, 0)),
        pl.BlockSpec((1,) + pe.shape[1:], lambda i: (i // _TILES_PER_BATCH, 0, 0, 0)),
        whole(m),
        whole(w0a), whole(b0a), whole(w0b), whole(b0b), whole(wh), whole(bh),
        whole(w1a), whole(b1a), whole(w1b), whole(b1b),
        whole(w3a), whole(b3a), whole(w3b), whole(b3b),
    ]
    out_spec = pl.BlockSpec((1, _ROWS_PER_TILE, _W),
                            lambda i: (i // _TILES_PER_BATCH,
                                       lax.rem(i, _TILES_PER_BATCH), 0))
    coarse, fine = pl.pallas_call(
        _tile_kernel,
        grid=grid,
        in_specs=in_specs,
        out_specs=[out_spec, out_spec],
        out_shape=[jax.ShapeDtypeStruct((_B, _H, _W), jnp.float32)] * 2,
    )(emb, pe, m, w0a, b0a, w0b, b0b, wh, bh,
      w1a, b1a, w1b, b1b, w3a, b3a, w3b, b3b)
    return (coarse.reshape(_B, 1, _H, _W), fine.reshape(_B, 1, _H, _W))


def kernel(image_embedding, image_pe, params, original_shape):
    del original_shape
    return _run(image_embedding, image_pe, params)
